# baseline (speedup 1.0000x reference)
import functools

import jax
import jax.numpy as jnp
from jax import lax
from jax.experimental import pallas as pl
from jax.experimental.pallas import tpu as pltpu
from jax.experimental.pallas import tpu_sc as plsc

NBITS = 4
NCODE = 16
NFLAT = 256
LANES = 16
HBINS = NFLAT * LANES


def _sc_hist_body(nw, per_w, chunk, pred_hbm, true_hbm, out_hbm,
                  pred_buf, true_buf, hist):
    c = lax.axis_index("c")
    s = lax.axis_index("s")
    wid = s * 2 + c

    lane = lax.iota(jnp.int32, LANES)
    base4 = lane * NBITS
    lane_base = lane * NFLAT
    ones = jnp.ones((LANES,), jnp.float32)
    zeros = jnp.zeros((LANES,), jnp.float32)

    for j in range(HBINS // LANES):
        hist[pl.ds(j * LANES, LANES)] = zeros

    groups = chunk // LANES
    nchunk = per_w // chunk

    def group_body(i, _):
        off = i * (LANES * NBITS)
        code = None
        for buf in (pred_buf, true_buf):
            for j in range(NBITS):
                bit = plsc.load_gather(buf, [base4 + (off + j)])
                code = bit if code is None else code * 2.0 + bit
        idx = lane_base + code.astype(jnp.int32)
        plsc.addupdate_scatter(hist, [idx], ones)
        return 0

    for ci in range(nchunk):
        off_words = (wid * per_w + ci * chunk) * NBITS
        pltpu.sync_copy(pred_hbm.at[pl.ds(off_words, chunk * NBITS)], pred_buf)
        pltpu.sync_copy(true_hbm.at[pl.ds(off_words, chunk * NBITS)], true_buf)
        lax.fori_loop(0, groups, group_body, 0)

    pltpu.sync_copy(hist, out_hbm.at[pl.ds(wid * HBINS, HBINS)])


def _tc_reduce_body(inv_total, p_ref, o_ref):
    o_ref[...] = p_ref[...].sum(axis=0, keepdims=True) * inv_total


@jax.jit
def kernel(y_pred, y_true):
    b = y_pred.shape[0]
    info = plsc.get_sparse_core_info()
    nw = info.num_cores * info.num_subcores
    per_w = b // nw
    chunk = min(per_w, 8192)

    mesh = plsc.VectorSubcoreMesh(core_axis_name="c", subcore_axis_name="s")
    sc_hist = pl.kernel(
        functools.partial(_sc_hist_body, nw, per_w, chunk),
        out_type=jax.ShapeDtypeStruct((nw * HBINS,), jnp.float32),
        mesh=mesh,
        scratch_types=[
            pltpu.VMEM((chunk * NBITS,), jnp.float32),
            pltpu.VMEM((chunk * NBITS,), jnp.float32),
            pltpu.VMEM((HBINS,), jnp.float32),
        ],
        compiler_params=pltpu.CompilerParams(needs_layout_passes=False),
    )
    partials = sc_hist(y_pred.reshape(-1), y_true.reshape(-1))

    reduce_call = pl.pallas_call(
        functools.partial(_tc_reduce_body, 1.0 / b),
        out_shape=jax.ShapeDtypeStruct((1, NFLAT), jnp.float32),
    )
    report = reduce_call(partials.reshape(nw * LANES, NFLAT))
    return report.reshape(NCODE, NCODE)


# speedup vs baseline: 1.0126x; 1.0126x over previous
import functools

import jax
import jax.numpy as jnp
import numpy as np
from jax import lax
from jax.experimental import pallas as pl
from jax.experimental import pallas as pl
from jax.experimental.pallas import tpu as pltpu
from jax.experimental.pallas import tpu_sc as plsc

NBITS = 4
NCODE = 16
NFLAT = 256
LANES = 16
HBINS = NFLAT * LANES
ROW = 128
SHOTS_PER_ROW = ROW // NBITS

ENC_BLOCK_ROWS = 1024
ENC_OUT_ROWS = ENC_BLOCK_ROWS // NBITS


def _make_encode_w():
    w = np.zeros((NBITS, ROW, ROW), np.float32)
    bitw = (8.0, 4.0, 2.0, 1.0)
    for q in range(NBITS):
        for j in range(SHOTS_PER_ROW):
            for k in range(NBITS):
                w[q, NBITS * j + k, SHOTS_PER_ROW * q + j] = bitw[k]
    return jnp.asarray(w)


def _tc_encode_body(xp_ref, xt_ref, w_ref, o_ref):
    acc = None
    for q in range(NBITS):
        sl = pl.ds(ENC_OUT_ROWS * q, ENC_OUT_ROWS)
        y = xp_ref[sl, :] * 16.0 + xt_ref[sl, :]
        t = jnp.dot(y, w_ref[q], preferred_element_type=jnp.float32)
        acc = t if acc is None else acc + t
    o_ref[...] = acc.astype(jnp.int32)


def _sc_hist_body(rows_per_tile, codes_hbm, out_hbm, buf, hist):
    c = lax.axis_index("c")
    s = lax.axis_index("s")
    wid = s * 2 + c

    lane_base = lax.iota(jnp.int32, LANES) * NFLAT
    ones = jnp.ones((LANES,), jnp.float32)
    zeros = jnp.zeros((LANES,), jnp.float32)

    pltpu.sync_copy(codes_hbm.at[pl.ds(wid * rows_per_tile, rows_per_tile)],
                    buf)

    for j in range(HBINS // LANES):
        hist[pl.ds(j * LANES, LANES)] = zeros

    def row_body(r, _):
        for cgrp in range(ROW // LANES):
            v = buf[r, pl.ds(cgrp * LANES, LANES)]
            plsc.addupdate_scatter(hist, [lane_base + v], ones)
        return 0

    lax.fori_loop(0, rows_per_tile, row_body, 0)

    pltpu.sync_copy(hist, out_hbm.at[pl.ds(wid * HBINS, HBINS)])


def _tc_reduce_body(inv_total, p_ref, o_ref):
    o_ref[...] = p_ref[...].sum(axis=0, keepdims=True) * inv_total


@jax.jit
def kernel(y_pred, y_true):
    b = y_pred.shape[0]
    info = plsc.get_sparse_core_info()
    nw = info.num_cores * info.num_subcores
    bit_rows = b * NBITS // ROW
    code_rows = b // ROW
    rows_per_tile = code_rows // nw
    grid = bit_rows // ENC_BLOCK_ROWS

    encode_call = pl.pallas_call(
        _tc_encode_body,
        grid=(grid,),
        in_specs=[
            pl.BlockSpec((ENC_BLOCK_ROWS, ROW), lambda g: (g, 0)),
            pl.BlockSpec((ENC_BLOCK_ROWS, ROW), lambda g: (g, 0)),
            pl.BlockSpec((NBITS, ROW, ROW), lambda g: (0, 0, 0)),
        ],
        out_specs=pl.BlockSpec((ENC_OUT_ROWS, ROW), lambda g: (g, 0)),
        out_shape=jax.ShapeDtypeStruct((code_rows, ROW), jnp.int32),
    )
    codes = encode_call(y_pred.reshape(bit_rows, ROW),
                        y_true.reshape(bit_rows, ROW),
                        _make_encode_w())

    mesh = plsc.VectorSubcoreMesh(core_axis_name="c", subcore_axis_name="s")
    sc_hist = pl.kernel(
        functools.partial(_sc_hist_body, rows_per_tile),
        out_type=jax.ShapeDtypeStruct((nw * HBINS,), jnp.float32),
        mesh=mesh,
        scratch_types=[
            pltpu.VMEM((rows_per_tile, ROW), jnp.int32),
            pltpu.VMEM((HBINS,), jnp.float32),
        ],
        compiler_params=pltpu.CompilerParams(needs_layout_passes=False,
                                             use_tc_tiling_on_sc=True),
    )
    partials = sc_hist(codes)

    reduce_call = pl.pallas_call(
        functools.partial(_tc_reduce_body, 1.0 / b),
        out_shape=jax.ShapeDtypeStruct((1, NFLAT), jnp.float32),
    )
    report = reduce_call(partials.reshape(nw * LANES, NFLAT))
    return report.reshape(NCODE, NCODE)


# speedup vs baseline: 87.7040x; 86.6093x over previous
import functools

import jax
import jax.numpy as jnp
from jax import lax
from jax.experimental import pallas as pl

NBITS = 4
NCODE = 16
CHUNK = 32768


def _hist_body(inv_total, nsteps, xp_ref, xt_ref, o_ref):
    g = pl.program_id(0)

    @pl.when(g == 0)
    def _init():
        o_ref[...] = jnp.zeros_like(o_ref)

    def encode(ref):
        acc = None
        for n in range(NBITS):
            bit = ref[n, 0]
            acc = bit if acc is None else acc * 2.0 + bit
        return acc

    cp = encode(xp_ref).astype(jnp.int32)
    ct = encode(xt_ref).astype(jnp.int32)
    io = lax.broadcasted_iota(jnp.int32, (NCODE, 1), 0)
    mp = (cp == io).astype(jnp.float32)
    mt = (ct == io).astype(jnp.float32)
    o_ref[...] += lax.dot_general(mp, mt, (((1,), (1,)), ((), ())),
                                  preferred_element_type=jnp.float32)

    @pl.when(g == nsteps - 1)
    def _norm():
        o_ref[...] = o_ref[...] * inv_total


@jax.jit
def kernel(y_pred, y_true):
    b = y_pred.shape[0]
    nsteps = b // CHUNK

    hist_call = pl.pallas_call(
        functools.partial(_hist_body, 1.0 / b, nsteps),
        grid=(nsteps,),
        in_specs=[
            pl.BlockSpec((NBITS, 1, 1, CHUNK), lambda g: (0, g, 0, 0)),
            pl.BlockSpec((NBITS, 1, 1, CHUNK), lambda g: (0, g, 0, 0)),
        ],
        out_specs=pl.BlockSpec((NCODE, NCODE), lambda g: (0, 0)),
        out_shape=jax.ShapeDtypeStruct((NCODE, NCODE), jnp.float32),
    )
    xp = jnp.transpose(y_pred, (1, 2, 0)).reshape(NBITS, nsteps, 1, CHUNK)
    xt = jnp.transpose(y_true, (1, 2, 0)).reshape(NBITS, nsteps, 1, CHUNK)
    return hist_call(xp, xt)
